# Initial kernel scaffold; baseline (speedup 1.0000x reference)
import jax
import jax.numpy as jnp
from jax.experimental import pallas as pl


def kernel(x, W1, b1, W2, b2, edge_index):
    raise NotImplementedError("write your pallas kernel here")


# speedup vs baseline: 13.8809x; 13.8809x over previous
import functools

import jax
import jax.numpy as jnp
from jax import lax
from jax.experimental import pallas as pl
from jax.experimental.pallas import tpu as pltpu
from jax.experimental.pallas import tpu_sc as plsc

N = 10000
NPAD = 10240
F_IN = 256
FH = 128
HID = 512
C = 64
E = 160000
NC = 2
NS = 16
K = 125

_MESH = plsc.VectorSubcoreMesh(
    core_axis_name="c", subcore_axis_name="s", num_cores=NC, num_subcores=NS)
_SC_PARAMS = pltpu.CompilerParams(use_tc_tiling_on_sc=False)

_f32 = jnp.float32


def _fill_zeros(buf, rows, width):
    for r in range(rows):
        for j in range(width // 16):
            buf[r, pl.ds(j * 16, 16)] = jnp.zeros((16,), _f32)


_DEG_W = 16
_DEG_CH = 40


@functools.partial(
    pl.kernel,
    out_type=jax.ShapeDtypeStruct((NC, NPAD, _DEG_W), _f32),
    mesh=_MESH,
    scratch_types=[
        pltpu.VMEM((_DEG_CH, K), jnp.int32),
        pltpu.VMEM((K, _DEG_W), _f32),
        pltpu.VMEM((128, _DEG_W), _f32),
        pltpu.VMEM_SHARED((NPAD, _DEG_W), _f32),
    ],
    compiler_params=_SC_PARAMS,
)
def _sc_deg(dst_hbm, out_hbm, didx_v, ones_v, zbuf, acc_sh):
    c = lax.axis_index("c")
    s = lax.axis_index("s")
    _fill_zeros(zbuf, 128, _DEG_W)
    for r in range(K):
        ones_v[r, pl.ds(0, 16)] = jnp.ones((16,), _f32)
    for j in range(5):
        pltpu.sync_copy(zbuf, acc_sh.at[pl.ds(s * 640 + j * 128, 128)])
    plsc.subcore_barrier()
    pltpu.sync_copy(dst_hbm.at[c, s], didx_v)

    def body(k, _):
        pltpu.sync_copy(ones_v, acc_sh.at[didx_v.at[k]], add=True)
        return _

    lax.fori_loop(0, _DEG_CH, body, 0)
    plsc.subcore_barrier()
    pltpu.sync_copy(acc_sh.at[pl.ds(s * 640, 640)],
                    out_hbm.at[c, pl.ds(s * 640, 640)])


def _make_sc_agg(width, nch, name):
    @functools.partial(
        pl.kernel,
        out_type=jax.ShapeDtypeStruct((NC, NPAD, width), _f32),
        mesh=_MESH,
        scratch_types=[
            pltpu.VMEM((nch, K), jnp.int32),
            pltpu.VMEM((nch, K), jnp.int32),
            pltpu.VMEM((K, width), _f32),
            pltpu.VMEM((128, width), _f32),
            pltpu.VMEM_SHARED((NPAD, width), _f32),
        ],
        compiler_params=_SC_PARAMS,
        name=name,
    )
    def agg(table_hbm, gidx_hbm, didx_hbm, out_hbm,
            gidx_v, didx_v, rows_v, zbuf, acc_sh):
        c = lax.axis_index("c")
        s = lax.axis_index("s")
        _fill_zeros(zbuf, 128, width)
        for j in range(5):
            pltpu.sync_copy(zbuf, acc_sh.at[pl.ds(s * 640 + j * 128, 128)])
        plsc.subcore_barrier()
        pltpu.sync_copy(gidx_hbm.at[c, s], gidx_v)
        pltpu.sync_copy(didx_hbm.at[c, s], didx_v)

        def body(k, _):
            pltpu.sync_copy(table_hbm.at[gidx_v.at[k]], rows_v)
            pltpu.sync_copy(rows_v, acc_sh.at[didx_v.at[k]], add=True)
            return _

        lax.fori_loop(0, nch, body, 0)
        plsc.subcore_barrier()
        pltpu.sync_copy(acc_sh.at[pl.ds(s * 640, 640)],
                        out_hbm.at[c, pl.ds(s * 640, 640)])

    return agg


_sc_agg1 = _make_sc_agg(64, 80, "sc_agg1")
_sc_agg2 = _make_sc_agg(C, 40, "sc_agg2")


def _tc_dis_body(degp_ref, dis_ref):
    deg = jnp.sum(degp_ref[...], axis=(0, 3)) * (1.0 / _DEG_W) + 1.0
    dis_ref[...] = lax.rsqrt(deg)


def _tc_dis(degp):
    return pl.pallas_call(
        _tc_dis_body,
        out_shape=jax.ShapeDtypeStruct((80, 128), _f32),
    )(degp)


def _tc_scale_body(x_ref, dis_ref, *out_refs):
    x1 = x_ref[...] * dis_ref[...]
    for q, o_ref in enumerate(out_refs):
        o_ref[...] = x1[:, q * 64:(q + 1) * 64]


def _tc_scale(x, dis_col):
    blk = 1000
    return pl.pallas_call(
        _tc_scale_body,
        grid=(N // blk,),
        in_specs=[
            pl.BlockSpec((blk, F_IN), lambda i: (i, 0)),
            pl.BlockSpec((blk, 1), lambda i: (i, 0)),
        ],
        out_specs=[pl.BlockSpec((blk, 64), lambda i: (i, 0))] * 4,
        out_shape=[jax.ShapeDtypeStruct((N, 64), _f32)] * 4,
    )(x, dis_col)


def _tc_mlp_body(a0_ref, a1_ref, a2_ref, a3_ref, x_ref, dis_ref, W1_ref,
                 b1_ref, W2_ref, p1_ref):
    dis = dis_ref[...]
    agg = jnp.concatenate(
        [a0_ref[...], a1_ref[...], a2_ref[...], a3_ref[...]], axis=1)
    ax = dis * agg + (dis * dis) * x_ref[...]
    h = jnp.maximum(
        jnp.dot(ax, W1_ref[...], preferred_element_type=_f32) + b1_ref[...],
        0.0)
    p = jnp.dot(h, W2_ref[...], preferred_element_type=_f32)
    p1_ref[...] = dis * p


def _tc_mlp(a0, a1, a2, a3, x, dis_col, W1, b1, W2):
    blk = 1000
    return pl.pallas_call(
        _tc_mlp_body,
        grid=(N // blk,),
        in_specs=[
            pl.BlockSpec((blk, 64), lambda i: (i, 0)),
            pl.BlockSpec((blk, 64), lambda i: (i, 0)),
            pl.BlockSpec((blk, 64), lambda i: (i, 0)),
            pl.BlockSpec((blk, 64), lambda i: (i, 0)),
            pl.BlockSpec((blk, F_IN), lambda i: (i, 0)),
            pl.BlockSpec((blk, 1), lambda i: (i, 0)),
            pl.BlockSpec((F_IN, HID), lambda i: (0, 0)),
            pl.BlockSpec((1, HID), lambda i: (0, 0)),
            pl.BlockSpec((HID, C), lambda i: (0, 0)),
        ],
        out_specs=pl.BlockSpec((blk, C), lambda i: (i, 0)),
        out_shape=jax.ShapeDtypeStruct((N, C), _f32),
    )(a0, a1, a2, a3, x, dis_col, W1, b1, W2)


def _tc_final_body(a0_ref, a1_ref, p1_ref, dis_ref, b2_ref, logp_ref, z_ref):
    dis = dis_ref[...]
    z = dis * (a0_ref[...] + a1_ref[...] + p1_ref[...]) + b2_ref[...]
    m = jnp.max(z, axis=1, keepdims=True)
    lse = jnp.log(jnp.sum(jnp.exp(z - m), axis=1, keepdims=True)) + m
    logp_ref[...] = z - lse
    z_ref[...] = z


def _tc_final(a0, a1, p1, dis_col, b2):
    blk = 1000
    return pl.pallas_call(
        _tc_final_body,
        grid=(N // blk,),
        in_specs=[
            pl.BlockSpec((blk, C), lambda i: (i, 0)),
            pl.BlockSpec((blk, C), lambda i: (i, 0)),
            pl.BlockSpec((blk, C), lambda i: (i, 0)),
            pl.BlockSpec((blk, 1), lambda i: (i, 0)),
            pl.BlockSpec((1, C), lambda i: (0, 0)),
        ],
        out_specs=[
            pl.BlockSpec((blk, C), lambda i: (i, 0)),
            pl.BlockSpec((blk, C), lambda i: (i, 0)),
        ],
        out_shape=[
            jax.ShapeDtypeStruct((N, C), _f32),
            jax.ShapeDtypeStruct((N, C), _f32),
        ],
    )(a0, a1, p1, dis_col, b2)


def kernel(x, W1, b1, W2, b2, edge_index):
    src = edge_index[0]
    dst = edge_index[1]

    dstA = dst.reshape(NC, NS, _DEG_CH, K)
    degp = _sc_deg(dstA)
    dis80 = _tc_dis(degp.reshape(NC, 80, 128, _DEG_W))
    dis_col = dis80.reshape(NPAD, 1)[:N]

    x1q = _tc_scale(x, dis_col)
    x1p = jnp.concatenate(x1q, axis=0)
    srcb = src.reshape(1, NS, 80, K)
    didx1 = jnp.broadcast_to(dst.reshape(1, NS, 80, K), (NC, NS, 80, K))
    offs = jnp.array([0, N], dtype=jnp.int32).reshape(NC, 1, 1, 1)
    aggA = _sc_agg1(x1p, srcb + offs, didx1)
    aggB = _sc_agg1(x1p, srcb + (offs + 2 * N), didx1)

    p1 = _tc_mlp(aggA[0, :N], aggA[1, :N], aggB[0, :N], aggB[1, :N],
                 x, dis_col, W1, b1.reshape(1, HID), W2)

    gidx2 = src.reshape(NC, NS, 40, K)
    didx2 = dst.reshape(NC, NS, 40, K)
    agg2 = _sc_agg2(p1, gidx2, didx2)

    logp, z = _tc_final(agg2[0, :N], agg2[1, :N], p1, dis_col,
                        b2.reshape(1, C))
    return (logp, z)


# speedup vs baseline: 20.8936x; 1.5052x over previous
import functools

import jax
import jax.numpy as jnp
from jax import lax
from jax.experimental import pallas as pl
from jax.experimental.pallas import tpu as pltpu
from jax.experimental.pallas import tpu_sc as plsc

N = 10000
NPAD = 10240
F_IN = 256
FH = 128
HID = 512
C = 64
E = 160000
NC = 2
NS = 16
K = 125

_MESH = plsc.VectorSubcoreMesh(
    core_axis_name="c", subcore_axis_name="s", num_cores=NC, num_subcores=NS)
_SC_PARAMS = pltpu.CompilerParams(use_tc_tiling_on_sc=False)

_f32 = jnp.float32


def _fill_zeros(buf, rows, width):
    for r in range(rows):
        for j in range(width // 16):
            buf[r, pl.ds(j * 16, 16)] = jnp.zeros((16,), _f32)


_DEG_W = 16
_DEG_CH = 40


@functools.partial(
    pl.kernel,
    out_type=jax.ShapeDtypeStruct((NC, NPAD, _DEG_W), _f32),
    mesh=_MESH,
    scratch_types=[
        pltpu.VMEM((_DEG_CH, K), jnp.int32),
        pltpu.VMEM((K, _DEG_W), _f32),
        pltpu.VMEM((128, _DEG_W), _f32),
        pltpu.VMEM_SHARED((NPAD, _DEG_W), _f32),
    ],
    compiler_params=_SC_PARAMS,
)
def _sc_deg(dst_hbm, out_hbm, didx_v, ones_v, zbuf, acc_sh):
    c = lax.axis_index("c")
    s = lax.axis_index("s")
    _fill_zeros(zbuf, 128, _DEG_W)
    for r in range(K):
        ones_v[r, pl.ds(0, 16)] = jnp.ones((16,), _f32)
    for j in range(5):
        pltpu.sync_copy(zbuf, acc_sh.at[pl.ds(s * 640 + j * 128, 128)])
    plsc.subcore_barrier()
    pltpu.sync_copy(dst_hbm.at[c, s], didx_v)

    def body(k, _):
        pltpu.sync_copy(ones_v, acc_sh.at[didx_v.at[k]], add=True)
        return _

    lax.fori_loop(0, _DEG_CH, body, 0)
    plsc.subcore_barrier()
    pltpu.sync_copy(acc_sh.at[pl.ds(s * 640, 640)],
                    out_hbm.at[c, pl.ds(s * 640, 640)])


_NBUF = 4


def _make_sc_agg(width, nch, nph, name):
    @functools.partial(
        pl.kernel,
        out_type=jax.ShapeDtypeStruct((nph, NC, NPAD, width), _f32),
        mesh=_MESH,
        scratch_types=(
            [pltpu.VMEM((nch, K), jnp.int32),
             pltpu.VMEM((nch, K), jnp.int32),
             pltpu.VMEM((128, width), _f32)]
            + [pltpu.VMEM((K, width), _f32) for _ in range(_NBUF)]
            + [pltpu.SemaphoreType.DMA for _ in range(_NBUF)]
            + [pltpu.VMEM_SHARED((NPAD, width), _f32)]
        ),
        compiler_params=_SC_PARAMS,
        name=name,
    )
    def agg(table_hbm, gidx_hbm, didx_hbm, out_hbm, gidx_v, didx_v, zbuf,
            *rest):
        bufs = rest[:_NBUF]
        sems = rest[_NBUF:2 * _NBUF]
        acc_sh = rest[2 * _NBUF]
        c = lax.axis_index("c")
        s = lax.axis_index("s")
        _fill_zeros(zbuf, 128, width)

        def zero_own():
            for j in range(5):
                pltpu.sync_copy(zbuf, acc_sh.at[pl.ds(s * 640 + j * 128, 128)])

        zero_own()
        pltpu.sync_copy(didx_hbm.at[c, s], didx_v)
        for q in range(nph):
            pltpu.sync_copy(gidx_hbm.at[q, c, s], gidx_v)
            plsc.subcore_barrier()
            for b in range(_NBUF - 1):
                pltpu.async_copy(table_hbm.at[gidx_v.at[b]], bufs[b], sems[b])

            def body(j, _):
                for b in range(_NBUF):
                    k = j * _NBUF + b
                    pltpu.make_async_copy(
                        table_hbm.at[gidx_v.at[k]], bufs[b], sems[b]).wait()
                    pltpu.sync_copy(bufs[b], acc_sh.at[didx_v.at[k]],
                                    add=True)
                    nxt = k + _NBUF - 1

                    @pl.when(nxt < nch)
                    def _start():
                        pltpu.async_copy(
                            table_hbm.at[gidx_v.at[nxt]],
                            bufs[(b + _NBUF - 1) % _NBUF],
                            sems[(b + _NBUF - 1) % _NBUF])
                return _

            lax.fori_loop(0, nch // _NBUF, body, 0)
            plsc.subcore_barrier()
            pltpu.sync_copy(acc_sh.at[pl.ds(s * 640, 640)],
                            out_hbm.at[q, c, pl.ds(s * 640, 640)])
            if q + 1 < nph:
                zero_own()

    return agg


_sc_agg1 = _make_sc_agg(64, 80, 2, "sc_agg1")
_sc_agg2 = _make_sc_agg(C, 40, 1, "sc_agg2")


def _tc_scale_body(degp_ref, x_ref, dis_ref, *out_refs):
    degs = jnp.sum(degp_ref[...], axis=0)
    deg = jnp.sum(degs, axis=1, keepdims=True) * (1.0 / _DEG_W) + 1.0
    dis = lax.rsqrt(deg)
    dis_ref[...] = dis
    x1 = x_ref[...] * dis
    for q, o_ref in enumerate(out_refs):
        o_ref[...] = x1[:, q * 64:(q + 1) * 64]


def _tc_scale(degp, x):
    blk = 1000
    return pl.pallas_call(
        _tc_scale_body,
        grid=(N // blk,),
        in_specs=[
            pl.BlockSpec((NC, blk, _DEG_W), lambda i: (0, i, 0)),
            pl.BlockSpec((blk, F_IN), lambda i: (i, 0)),
        ],
        out_specs=[pl.BlockSpec((blk, 1), lambda i: (i, 0))]
        + [pl.BlockSpec((blk, 64), lambda i: (i, 0))] * 4,
        out_shape=[jax.ShapeDtypeStruct((N, 1), _f32)]
        + [jax.ShapeDtypeStruct((N, 64), _f32)] * 4,
    )(degp, x)


def _tc_mlp_body(a0_ref, a1_ref, a2_ref, a3_ref, x_ref, dis_ref, W1_ref,
                 b1_ref, W2_ref, p1_ref):
    dis = dis_ref[...]
    agg = jnp.concatenate(
        [a0_ref[...], a1_ref[...], a2_ref[...], a3_ref[...]], axis=1)
    ax = dis * agg + (dis * dis) * x_ref[...]
    h = jnp.maximum(
        jnp.dot(ax, W1_ref[...], preferred_element_type=_f32) + b1_ref[...],
        0.0)
    p = jnp.dot(h, W2_ref[...], preferred_element_type=_f32)
    p1_ref[...] = dis * p


def _tc_mlp(a0, a1, a2, a3, x, dis_col, W1, b1, W2):
    blk = 1000
    return pl.pallas_call(
        _tc_mlp_body,
        grid=(N // blk,),
        in_specs=[
            pl.BlockSpec((blk, 64), lambda i: (i, 0)),
            pl.BlockSpec((blk, 64), lambda i: (i, 0)),
            pl.BlockSpec((blk, 64), lambda i: (i, 0)),
            pl.BlockSpec((blk, 64), lambda i: (i, 0)),
            pl.BlockSpec((blk, F_IN), lambda i: (i, 0)),
            pl.BlockSpec((blk, 1), lambda i: (i, 0)),
            pl.BlockSpec((F_IN, HID), lambda i: (0, 0)),
            pl.BlockSpec((1, HID), lambda i: (0, 0)),
            pl.BlockSpec((HID, C), lambda i: (0, 0)),
        ],
        out_specs=pl.BlockSpec((blk, C), lambda i: (i, 0)),
        out_shape=jax.ShapeDtypeStruct((N, C), _f32),
    )(a0, a1, a2, a3, x, dis_col, W1, b1, W2)


def _tc_final_body(a0_ref, a1_ref, p1_ref, dis_ref, b2_ref, logp_ref, z_ref):
    dis = dis_ref[...]
    z = dis * (a0_ref[...] + a1_ref[...] + p1_ref[...]) + b2_ref[...]
    m = jnp.max(z, axis=1, keepdims=True)
    lse = jnp.log(jnp.sum(jnp.exp(z - m), axis=1, keepdims=True)) + m
    logp_ref[...] = z - lse
    z_ref[...] = z


def _tc_final(a0, a1, p1, dis_col, b2):
    blk = 1000
    return pl.pallas_call(
        _tc_final_body,
        grid=(N // blk,),
        in_specs=[
            pl.BlockSpec((blk, C), lambda i: (i, 0)),
            pl.BlockSpec((blk, C), lambda i: (i, 0)),
            pl.BlockSpec((blk, C), lambda i: (i, 0)),
            pl.BlockSpec((blk, 1), lambda i: (i, 0)),
            pl.BlockSpec((1, C), lambda i: (0, 0)),
        ],
        out_specs=[
            pl.BlockSpec((blk, C), lambda i: (i, 0)),
            pl.BlockSpec((blk, C), lambda i: (i, 0)),
        ],
        out_shape=[
            jax.ShapeDtypeStruct((N, C), _f32),
            jax.ShapeDtypeStruct((N, C), _f32),
        ],
    )(a0, a1, p1, dis_col, b2)


def kernel(x, W1, b1, W2, b2, edge_index):
    src = edge_index[0]
    dst = edge_index[1]

    dstA = dst.reshape(NC, NS, _DEG_CH, K)
    degp = _sc_deg(dstA)

    dis_col, *x1q = _tc_scale(degp, x)
    x1p = jnp.concatenate(x1q, axis=0)
    srcb = src.reshape(1, 1, NS, 80, K)
    didx1 = jnp.broadcast_to(dst.reshape(1, NS, 80, K), (NC, NS, 80, K))
    offs = (jnp.arange(4, dtype=jnp.int32) * N).reshape(2, NC, 1, 1, 1)
    agg1 = _sc_agg1(x1p, srcb + offs, didx1)

    p1 = _tc_mlp(agg1[0, 0, :N], agg1[0, 1, :N], agg1[1, 0, :N],
                 agg1[1, 1, :N], x, dis_col, W1, b1.reshape(1, HID),
                 W2)

    gidx2 = src.reshape(1, NC, NS, 40, K)
    didx2 = dst.reshape(NC, NS, 40, K)
    agg2 = _sc_agg2(p1, gidx2, didx2)

    logp, z = _tc_final(agg2[0, 0, :N], agg2[0, 1, :N], p1, dis_col,
                        b2.reshape(1, C))
    return (logp, z)
